# baseline (speedup 1.0000x reference)
import functools

import jax
import jax.numpy as jnp
import numpy as np
from jax import lax
from jax.experimental import pallas as pl
from jax.experimental.pallas import tpu as pltpu
from jax.experimental.pallas import tpu_sc as plsc

N_ALPHA = 16
N_DEPTH = 12
ALPHA_MIN = -1.5707963267948966
ALPHA_RANGE = 3.141592653589793 + 1e-8
LD_MIN = -3.0
LD_RANGE = 8.0 + 1e-8
MAX_DELTA = 0.5

NUM_WORKERS = 32
LANES = 16

_ASIN_C = (
    1.570787800976358,
    -0.2141239403279347,
    0.08466461022358122,
    -0.035754589337180405,
    0.008648205906633571,
)


def _vsqrt(u):
    i = plsc.bitcast(u, jnp.int32)
    i = jnp.int32(0x5F3759DF) - lax.shift_right_logical(i, 1)
    r = plsc.bitcast(i, jnp.float32)
    half_u = 0.5 * u
    for _ in range(2):
        r = r * (1.5 - half_u * r * r)
    return u * r


def _alpha_grid_coord(y):
    k = (N_ALPHA - 1) / ALPHA_RANGE
    t = jnp.abs(y)
    u = jnp.maximum(1.0 - t, 1e-14)
    sq = _vsqrt(u)
    p = jnp.full_like(t, _ASIN_C[4])
    for c in (_ASIN_C[3], _ASIN_C[2], _ASIN_C[1], _ASIN_C[0]):
        p = p * t + c
    q = (sq * p) * k
    q = jnp.where(t >= 1.0, 0.0, q)
    return jnp.where(y < 0.0, q, float(N_ALPHA - 1) - q)


_BSPLINE_M = np.array(
    [[1 / 6, 4 / 6, 1 / 6, 0.0],
     [-1 / 2, 0.0, 1 / 2, 0.0],
     [1 / 2, -1.0, 1 / 2, 0.0],
     [-1 / 6, 1 / 2, -1 / 2, 1 / 6]], dtype=np.float64)

_W_FLAT = np.einsum(
    "pi,qj->ijpq", _BSPLINE_M, _BSPLINE_M).reshape(-1).astype(np.float32)

N_WIN = (N_ALPHA - 3) * (N_DEPTH - 3)
TROW = 17
TLEN_C = ((N_WIN * TROW + 15) // 16) * 16
T2ROW = TLEN_C + 8
T2LEN = 7 * T2ROW + TLEN_C


def _make_sc_call(n, chunk):
    rays_per_tile = n // NUM_WORKERS
    chunks_per_tile = rays_per_tile // chunk
    assert chunks_per_tile % 2 == 0
    n_pairs = chunks_per_tile // 2
    groups_per_chunk = chunk // LANES
    mesh = plsc.VectorSubcoreMesh(core_axis_name="c", subcore_axis_name="s")

    @functools.partial(
        pl.kernel,
        mesh=mesh,
        compiler_params=pltpu.CompilerParams(needs_layout_passes=False),
        out_type=jax.ShapeDtypeStruct((n,), jnp.float32),
        scratch_types=[
            pltpu.VMEM((N_ALPHA * N_DEPTH,), jnp.float32),
            pltpu.VMEM((256,), jnp.float32),
            pltpu.VMEM((TLEN_C,), jnp.float32),
            pltpu.VMEM((T2LEN,), jnp.float32),
            pltpu.VMEM((2 * chunk,), jnp.float32),
            pltpu.VMEM((2 * chunk,), jnp.float32),
            pltpu.VMEM((2 * chunk,), jnp.float32),
            pltpu.SemaphoreType.DMA,
            pltpu.SemaphoreType.DMA,
            pltpu.SemaphoreType.DMA,
            pltpu.SemaphoreType.DMA,
        ],
    )
    def sc_call(y_hbm, ld_hbm, cp_hbm, w_hbm, out_hbm, table_v, w_v, t_v,
                t2_v, y_v, ld_v, out_v, sem_in0, sem_in1, sem_out0, sem_out1):
        wid = lax.axis_index("s") * 2 + lax.axis_index("c")
        base = wid * rays_per_tile
        sem_in = (sem_in0, sem_in1)
        sem_out = (sem_out0, sem_out1)
        pltpu.sync_copy(cp_hbm, table_v)
        pltpu.sync_copy(w_hbm, w_v)
        lane = lax.iota(jnp.int32, LANES)

        def build_win(k, carry):
            a0 = k // (N_DEPTH - 3)
            d0 = k - a0 * (N_DEPTH - 3)
            cb = a0 * N_DEPTH + d0
            coef = jnp.zeros((LANES,), jnp.float32)
            for i in range(4):
                for j in range(4):
                    idx = jnp.broadcast_to(cb + i * N_DEPTH + j, (LANES,))
                    c_b = plsc.load_gather(table_v, [idx])
                    wv = w_v[pl.ds(LANES * (4 * i + j), LANES)]
                    coef = coef + wv * c_b
            plsc.store_scatter(t_v, [lane + k * TROW], coef)
            return carry

        lax.fori_loop(0, N_WIN, build_win, 0)

        def build_rep(g, carry):
            for r in range(8):
                srcr = jnp.minimum(lane + (g * LANES + r), TLEN_C - 1)
                valsr = plsc.load_gather(t_v, [srcr])
                plsc.store_scatter(t2_v, [lane + (g * LANES + r * T2ROW)],
                                   valsr)
            return carry

        lax.fori_loop(0, TLEN_C // LANES, build_rep, 0)

        def in_copies(c, b):
            row0 = base + c * chunk
            return (
                pltpu.make_async_copy(
                    y_hbm.at[pl.ds(row0, chunk)],
                    y_v.at[pl.ds(b * chunk, chunk)], sem_in[b]),
                pltpu.make_async_copy(
                    ld_hbm.at[pl.ds(row0, chunk)],
                    ld_v.at[pl.ds(b * chunk, chunk)], sem_in[b]),
            )

        def out_copy(c, b):
            row0 = base + c * chunk
            return pltpu.make_async_copy(
                out_v.at[pl.ds(b * chunk, chunk)],
                out_hbm.at[pl.ds(row0, chunk)], sem_out[b])

        for b in range(2):
            for cp in in_copies(b, b):
                cp.start()

        def do_pair(i, carry):
            for b in range(2):
                c = 2 * i + b
                for cp in in_copies(c, b):
                    cp.wait()

                @pl.when(i > 0)
                def _wait_prev_scatter():
                    out_copy(c - 2, b).wait()

                boff = b * chunk

                @plsc.parallel_loop(0, groups_per_chunk, unroll=32)
                def do_group(g):
                    off = boff + g * LANES
                    y = y_v[pl.ds(off, LANES)]
                    ld = ld_v[pl.ds(off, LANES)]

                    a_idx = _alpha_grid_coord(y)
                    d_idx = jnp.clip(
                        (ld - LD_MIN) * ((N_DEPTH - 1) / LD_RANGE),
                        0.0, float(N_DEPTH - 1))
                    fa = a_idx.astype(jnp.int32)
                    fd = d_idx.astype(jnp.int32)
                    a_loc = a_idx - fa.astype(jnp.float32)
                    d_loc = d_idx - fd.astype(jnp.float32)
                    a_start = jnp.clip(fa - 1, 0, N_ALPHA - 4)
                    d_start = jnp.clip(fd - 1, 0, N_DEPTH - 4)

                    flat0 = (a_start * (N_DEPTH - 3) + d_start) * TROW
                    cvs = []
                    for p in range(16):
                        sofs = T2ROW * (p & 7) + (p & ~7)
                        cvs.append(plsc.load_gather(
                            t2_v.at[pl.ds(sofs, T2LEN - sofs)], [flat0]))
                    rows = []
                    for pa in range(4):
                        c0, c1, c2, c3 = cvs[4 * pa:4 * pa + 4]
                        rows.append(
                            ((c3 * d_loc + c2) * d_loc + c1) * d_loc + c0)
                    acc = (((rows[3] * a_loc + rows[2]) * a_loc + rows[1])
                           * a_loc + rows[0])

                    res = jnp.clip(acc, -MAX_DELTA, MAX_DELTA)
                    out_v[pl.ds(off, LANES)] = res

                @pl.when(i < n_pairs - 1)
                def _prefetch_next():
                    for cp in in_copies(c + 2, b):
                        cp.start()

                out_copy(c, b).start()
            return carry

        lax.fori_loop(0, n_pairs, do_pair, 0)
        for b in range(2):
            out_copy(chunks_per_tile - 2 + b, b).wait()

    return sc_call


def kernel(ray_dirs, log_depth, control_points):
    n = ray_dirs.shape[0]
    ray_y = lax.squeeze(lax.slice(ray_dirs, (0, 1), (n, 2)), (1,))
    cp_flat = control_points.reshape(-1)
    w_flat = jnp.asarray(_W_FLAT)
    sc_call = _make_sc_call(n, 8192)
    return sc_call(ray_y, log_depth, cp_flat, w_flat)


# speedup vs baseline: 1.0146x; 1.0146x over previous
import functools

import jax
import jax.numpy as jnp
import numpy as np
from jax import lax
from jax.experimental import pallas as pl
from jax.experimental.pallas import tpu as pltpu
from jax.experimental.pallas import tpu_sc as plsc

N_ALPHA = 16
N_DEPTH = 12
ALPHA_MIN = -1.5707963267948966
ALPHA_RANGE = 3.141592653589793 + 1e-8
LD_MIN = -3.0
LD_RANGE = 8.0 + 1e-8
MAX_DELTA = 0.5

NUM_WORKERS = 32
LANES = 16

_ASIN_C = (
    1.570787800976358,
    -0.2141239403279347,
    0.08466461022358122,
    -0.035754589337180405,
    0.008648205906633571,
)


def _vsqrt(u):
    i = plsc.bitcast(u, jnp.int32)
    i = jnp.int32(0x5F3759DF) - lax.shift_right_logical(i, 1)
    r = plsc.bitcast(i, jnp.float32)
    half_u = 0.5 * u
    for _ in range(2):
        r = r * (1.5 - half_u * r * r)
    return u * r


def _alpha_grid_coord(y):
    k = (N_ALPHA - 1) / ALPHA_RANGE
    t = jnp.abs(y)
    u = jnp.maximum(1.0 - t, 1e-14)
    sq = _vsqrt(u)
    p = jnp.full_like(t, _ASIN_C[4])
    for c in (_ASIN_C[3], _ASIN_C[2], _ASIN_C[1], _ASIN_C[0]):
        p = p * t + c
    q = (sq * p) * k
    q = jnp.where(t >= 1.0, 0.0, q)
    return jnp.where(y < 0.0, q, float(N_ALPHA - 1) - q)


_BSPLINE_M = np.array(
    [[1 / 6, 4 / 6, 1 / 6, 0.0],
     [-1 / 2, 0.0, 1 / 2, 0.0],
     [1 / 2, -1.0, 1 / 2, 0.0],
     [-1 / 6, 1 / 2, -1 / 2, 1 / 6]], dtype=np.float64)

_W_FLAT = np.einsum(
    "pi,qj->ijpq", _BSPLINE_M, _BSPLINE_M).reshape(-1).astype(np.float32)

N_WIN = (N_ALPHA - 3) * (N_DEPTH - 3)
TROW = 17
TLEN_C = ((N_WIN * TROW + 15) // 16) * 16
T2ROW = TLEN_C + 8
T2LEN = 7 * T2ROW + TLEN_C


def _make_sc_call(n, chunk):
    rays_per_tile = n // NUM_WORKERS
    chunks_per_tile = rays_per_tile // chunk
    assert chunks_per_tile % 2 == 0
    n_pairs = chunks_per_tile // 2
    groups_per_chunk = chunk // LANES
    mesh = plsc.VectorSubcoreMesh(core_axis_name="c", subcore_axis_name="s")

    @functools.partial(
        pl.kernel,
        mesh=mesh,
        compiler_params=pltpu.CompilerParams(needs_layout_passes=False),
        out_type=jax.ShapeDtypeStruct((n,), jnp.float32),
        scratch_types=[
            pltpu.VMEM((N_ALPHA * N_DEPTH,), jnp.float32),
            pltpu.VMEM((256,), jnp.float32),
            pltpu.VMEM((TLEN_C,), jnp.float32),
            pltpu.VMEM((T2LEN,), jnp.float32),
            pltpu.VMEM((2 * chunk,), jnp.float32),
            pltpu.VMEM((2 * chunk,), jnp.float32),
            pltpu.VMEM((2 * chunk,), jnp.float32),
            pltpu.SemaphoreType.DMA,
            pltpu.SemaphoreType.DMA,
            pltpu.SemaphoreType.DMA,
            pltpu.SemaphoreType.DMA,
        ],
    )
    def sc_call(y_hbm, ld_hbm, cp_hbm, w_hbm, out_hbm, table_v, w_v, t_v,
                t2_v, y_v, ld_v, out_v, sem_in0, sem_in1, sem_out0, sem_out1):
        wid = lax.axis_index("s") * 2 + lax.axis_index("c")
        base = wid * rays_per_tile
        sem_in = (sem_in0, sem_in1)
        sem_out = (sem_out0, sem_out1)
        pltpu.sync_copy(cp_hbm, table_v)
        pltpu.sync_copy(w_hbm, w_v)
        lane = lax.iota(jnp.int32, LANES)

        def build_win(k, carry):
            a0 = k // (N_DEPTH - 3)
            d0 = k - a0 * (N_DEPTH - 3)
            cb = a0 * N_DEPTH + d0
            coef = jnp.zeros((LANES,), jnp.float32)
            for i in range(4):
                for j in range(4):
                    idx = jnp.broadcast_to(cb + i * N_DEPTH + j, (LANES,))
                    c_b = plsc.load_gather(table_v, [idx])
                    wv = w_v[pl.ds(LANES * (4 * i + j), LANES)]
                    coef = coef + wv * c_b
            plsc.store_scatter(t_v, [lane + k * TROW], coef)
            return carry

        lax.fori_loop(0, N_WIN, build_win, 0)

        def build_rep(g, carry):
            for r in range(8):
                srcr = jnp.minimum(lane + (g * LANES + r), TLEN_C - 1)
                valsr = plsc.load_gather(t_v, [srcr])
                plsc.store_scatter(t2_v, [lane + (g * LANES + r * T2ROW)],
                                   valsr)
            return carry

        lax.fori_loop(0, TLEN_C // LANES, build_rep, 0)

        def in_copies(c, b):
            row0 = base + c * chunk
            return (
                pltpu.make_async_copy(
                    y_hbm.at[pl.ds(row0, chunk)],
                    y_v.at[pl.ds(b * chunk, chunk)], sem_in[b]),
                pltpu.make_async_copy(
                    ld_hbm.at[pl.ds(row0, chunk)],
                    ld_v.at[pl.ds(b * chunk, chunk)], sem_in[b]),
            )

        def out_copy(c, b):
            row0 = base + c * chunk
            return pltpu.make_async_copy(
                out_v.at[pl.ds(b * chunk, chunk)],
                out_hbm.at[pl.ds(row0, chunk)], sem_out[b])

        for b in range(2):
            for cp in in_copies(b, b):
                cp.start()

        def do_pair(i, carry):
            for b in range(2):
                c = 2 * i + b
                for cp in in_copies(c, b):
                    cp.wait()

                @pl.when(i > 0)
                def _wait_prev_scatter():
                    out_copy(c - 2, b).wait()

                boff = b * chunk

                @plsc.parallel_loop(0, groups_per_chunk, unroll=16)
                def do_group(g):
                    off = boff + g * LANES
                    y = y_v[pl.ds(off, LANES)]
                    ld = ld_v[pl.ds(off, LANES)]

                    a_idx = _alpha_grid_coord(y)
                    d_idx = jnp.clip(
                        (ld - LD_MIN) * ((N_DEPTH - 1) / LD_RANGE),
                        0.0, float(N_DEPTH - 1))
                    fa = a_idx.astype(jnp.int32)
                    fd = d_idx.astype(jnp.int32)
                    a_loc = a_idx - fa.astype(jnp.float32)
                    d_loc = d_idx - fd.astype(jnp.float32)
                    a_start = jnp.clip(fa - 1, 0, N_ALPHA - 4)
                    d_start = jnp.clip(fd - 1, 0, N_DEPTH - 4)

                    flat0 = (a_start * (N_DEPTH - 3) + d_start) * TROW
                    cvs = []
                    for p in range(16):
                        sofs = T2ROW * (p & 7) + (p & ~7)
                        cvs.append(plsc.load_gather(
                            t2_v.at[pl.ds(sofs, T2LEN - sofs)], [flat0]))
                    rows = []
                    for pa in range(4):
                        c0, c1, c2, c3 = cvs[4 * pa:4 * pa + 4]
                        rows.append(
                            ((c3 * d_loc + c2) * d_loc + c1) * d_loc + c0)
                    acc = (((rows[3] * a_loc + rows[2]) * a_loc + rows[1])
                           * a_loc + rows[0])

                    res = jnp.clip(acc, -MAX_DELTA, MAX_DELTA)
                    out_v[pl.ds(off, LANES)] = res

                @pl.when(i < n_pairs - 1)
                def _prefetch_next():
                    for cp in in_copies(c + 2, b):
                        cp.start()

                out_copy(c, b).start()
            return carry

        lax.fori_loop(0, n_pairs, do_pair, 0)
        for b in range(2):
            out_copy(chunks_per_tile - 2 + b, b).wait()

    return sc_call


def kernel(ray_dirs, log_depth, control_points):
    n = ray_dirs.shape[0]
    ray_y = lax.squeeze(lax.slice(ray_dirs, (0, 1), (n, 2)), (1,))
    cp_flat = control_points.reshape(-1)
    w_flat = jnp.asarray(_W_FLAT)
    sc_call = _make_sc_call(n, 8192)
    return sc_call(ray_y, log_depth, cp_flat, w_flat)
